# Initial kernel scaffold; baseline (speedup 1.0000x reference)
import jax
import jax.numpy as jnp
from jax.experimental import pallas as pl


def kernel(batch, X, H, edge_index, te, e3_weight, ln_gamma, ln_beta, Wm1, bm1, Wm2, bm2, Wx1, bx1, Wx2, bx2, Wc1, bc1, Wc2, bc2):
    raise NotImplementedError("write your pallas kernel here")


# speedup vs baseline: 34.4144x; 34.4144x over previous
import jax
import jax.numpy as jnp
from jax import lax
from jax.experimental import pallas as pl
from jax.experimental.pallas import tpu as pltpu
from jax.experimental.pallas import tpu_sc as plsc

F32 = jnp.float32

N = 50000
E = 800000
B = 256
K = 16
HD = 64
XF = 3 * K

NB = 2000
GN = N // NB
EB = 2000
GE = E // EB

NC = 2
NS = 16
NW = NC * NS
EPW = E // NW
C = 128
NCHUNK_G = -(-EPW // C)
G_LAST = EPW - C

EPT = E // NS
NCHUNK_S = EPT // C
S_TAIL = EPT - NCHUNK_S * C

SROW = 128

NP = 50176
NR = NP // 2
NHR = NR // NC
NHN = NHR * 2
TROW = NHR
ACC_H = NHR + 8
CPR = NHR // C
NCPT = (CPR + NS - 1) // NS


def _silu(x):
    return x / (1.0 + jnp.exp(-x))


def _onehot(b):
    return (b[:, None] == lax.broadcasted_iota(jnp.int32, (b.shape[0], B), 1)
            ).astype(F32)


def _segsum(oh, x):
    return lax.dot_general(oh, x, (((0,), (0,)), ((), ())),
                           preferred_element_type=F32)


def n1_body(b_ref, x_ref, out_ref):
    i = pl.program_id(0)
    oh = _onehot(b_ref[0, 0, :])
    sums = _segsum(oh, x_ref[...])
    cnts = _segsum(oh, jnp.ones((NB, K), F32))
    blk = jnp.concatenate([sums, cnts], axis=1)

    @pl.when(i == 0)
    def _():
        out_ref[...] = blk

    @pl.when(i > 0)
    def _():
        out_ref[...] += blk


def n2_body(b_ref, x_ref, g1_ref, out_ref):
    i = pl.program_id(0)
    oh = _onehot(b_ref[0, 0, :])
    g1 = g1_ref[...]
    cnt16 = jnp.maximum(g1[:, XF:], 1.0)
    cnt48 = jnp.concatenate([cnt16, cnt16, cnt16], axis=1)
    mean = g1[:, :XF] / cnt48
    xc = x_ref[...] - jnp.dot(oh, mean, preferred_element_type=F32)
    x0, x1, x2 = xc[:, :K], xc[:, K:2 * K], xc[:, 2 * K:]
    nrm = jnp.sqrt(x0 * x0 + x1 * x1 + x2 * x2)
    blk = _segsum(oh, nrm)

    @pl.when(i == 0)
    def _():
        out_ref[...] = blk

    @pl.when(i > 0)
    def _():
        out_ref[...] += blk


def n3_body(b_ref, x_ref, h_ref, te_ref, g1_ref, g2_ref, wm1_ref, bm1_ref,
            lng_ref, lnb_ref, e3_ref, xn_ref, s_ref, t_ref, g3_ref):
    i = pl.program_id(0)
    oh = _onehot(b_ref[0, 0, :])
    g1 = g1_ref[...]
    cnt16 = jnp.maximum(g1[:, XF:], 1.0)
    cnt48 = jnp.concatenate([cnt16, cnt16, cnt16], axis=1)
    mean = g1[:, :XF] / cnt48
    xc = x_ref[...] - jnp.dot(oh, mean, preferred_element_type=F32)
    x0, x1, x2 = xc[:, :K], xc[:, K:2 * K], xc[:, 2 * K:]
    mn = g2_ref[...] / cnt16
    denom = jnp.dot(oh, mn, preferred_element_type=F32) + 1e-5
    e3 = e3_ref[...]
    xn = jnp.concatenate(
        [e3 * x0 / denom, e3 * x1 / denom, e3 * x2 / denom], axis=1)
    xn_ref[...] = xn

    h = h_ref[...]
    mu = jnp.mean(h, axis=1, keepdims=True)
    var = jnp.mean((h - mu) ** 2, axis=1, keepdims=True)
    hn = (h - mu) / jnp.sqrt(var + 1e-5) * lng_ref[...] + lnb_ref[...]

    wm1 = wm1_ref[...]
    w_t = wm1[0:HD, :]
    w_s = wm1[HD:2 * HD, :]
    w_te = wm1[2 * HD + K:, :]
    te2 = jnp.dot(te_ref[...], w_te, preferred_element_type=F32)
    a = (jnp.dot(hn, w_s, preferred_element_type=F32)
         + jnp.dot(oh, te2, preferred_element_type=F32)
         + bm1_ref[...])
    hwt = jnp.dot(hn, w_t, preferred_element_type=F32)
    pad = jnp.zeros((NB, SROW - HD - XF), F32)
    s_ref[...] = jnp.concatenate([a, xn, pad], axis=1)
    t_ref[...] = jnp.concatenate([hwt, xn, pad], axis=1)

    blk = _segsum(oh, xn)

    @pl.when(i == 0)
    def _():
        g3_ref[...] = blk

    @pl.when(i > 0)
    def _():
        g3_ref[...] += blk


def e1_body(src_ref, tgt_ref, gs_ref, gt_ref, g1_ref, g3_ref, wd_ref,
            wm2_ref, bm2_ref, wx1_ref, bx1_ref, wx2_ref, bx2_ref, wc1_ref,
            bc1_ref, wc2_ref, bc2_ref, out_ref):
    gs = gs_ref[...]
    gt = gt_ref[...]
    xs0, xs1, xs2 = (gs[:, HD:HD + K], gs[:, HD + K:HD + 2 * K],
                     gs[:, HD + 2 * K:HD + 3 * K])
    xt0, xt1, xt2 = (gt[:, HD:HD + K], gt[:, HD + K:HD + 2 * K],
                     gt[:, HD + 2 * K:HD + 3 * K])
    r0, r1, r2 = xs0 - xt0, xs1 - xt1, xs2 - xt2
    rd = r0 * r0 + r1 * r1 + r2 * r2

    pre1 = (gs[:, :HD] + gt[:, :HD]
            + jnp.dot(rd, wd_ref[...], preferred_element_type=F32))
    mij = (jnp.dot(_silu(pre1), wm2_ref[...], preferred_element_type=F32)
           + bm2_ref[...])
    hx = _silu(jnp.dot(mij, wx1_ref[...], preferred_element_type=F32)
               + bx1_ref[...])
    wx = jnp.clip(jnp.dot(hx, wx2_ref[...], preferred_element_type=F32)
                  + bx2_ref[...], -10.0, 10.0)
    hc = _silu(jnp.dot(mij, wc1_ref[...], preferred_element_type=F32)
               + bc1_ref[...])
    wc = jnp.clip(jnp.dot(hc, wc2_ref[...], preferred_element_type=F32)
                  + bc2_ref[...], -10.0, 10.0)

    inv = 1.0 / (1.0 + jnp.sqrt(rd + 1e-8))

    g1 = g1_ref[...]
    cnt16 = jnp.maximum(g1[:, XF:], 1.0)
    cnt48 = jnp.concatenate([cnt16, cnt16, cnt16], axis=1)
    m = g3_ref[...] / cnt48
    ohs = (src_ref[0, 0, :][:, None]
           == lax.broadcasted_iota(jnp.int32, (EB, B), 1)).astype(F32)
    oht = (tgt_ref[0, 0, :][:, None]
           == lax.broadcasted_iota(jnp.int32, (EB, B), 1)).astype(F32)
    ps = jnp.dot(ohs, m, preferred_element_type=F32)
    pt = jnp.dot(oht, m, preferred_element_type=F32)
    cs0 = xs0 - ps[:, :K]
    cs1 = xs1 - ps[:, K:2 * K]
    cs2 = xs2 - ps[:, 2 * K:]
    ct0 = xt0 - pt[:, :K]
    ct1 = xt1 - pt[:, K:2 * K]
    ct2 = xt2 - pt[:, 2 * K:]
    cr0 = cs1 * ct2 - cs2 * ct1
    cr1 = cs2 * ct0 - cs0 * ct2
    cr2 = cs0 * ct1 - cs1 * ct0
    cinv = 1.0 / (1.0 + jnp.sqrt(cr0 * cr0 + cr1 * cr1 + cr2 * cr2))
    o0 = r0 * inv * wx + cr0 * cinv * wc
    o1 = r1 * inv * wx + cr1 * cinv * wc
    o2 = r2 * inv * wx + cr2 * cinv * wc
    o = jnp.concatenate([o0, o1, o2], axis=1)
    par = (tgt_ref[0, 0, :] % 2).astype(F32)[:, None]
    z = jnp.zeros((EB, 16), F32)
    out_ref[...] = jnp.concatenate(
        [o * (1.0 - par), z, o * par, z], axis=1)


def gather_sc_body(s_hbm, t_hbm, src_hbm, tgt_hbm, gs_hbm, gt_hbm,
                   idx_v, rows_v, sem):
    wid = lax.axis_index("s") * NC + lax.axis_index("c")
    wbase = wid * EPW

    def chunk(j, carry):
        cb = wbase + jnp.minimum(j * C, G_LAST)
        pltpu.sync_copy(src_hbm.at[pl.ds(cb, C)], idx_v)
        pltpu.async_copy(s_hbm.at[idx_v], rows_v, sem).wait()
        pltpu.sync_copy(rows_v, gs_hbm.at[pl.ds(cb, C)])
        pltpu.sync_copy(tgt_hbm.at[pl.ds(cb, C)], idx_v)
        pltpu.async_copy(t_hbm.at[idx_v], rows_v, sem).wait()
        pltpu.sync_copy(rows_v, gt_hbm.at[pl.ds(cb, C)])
        return carry

    lax.fori_loop(0, NCHUNK_G, chunk, 0)


def scatter_sc_body(ct_hbm, tgt_hbm, xnp_hbm, outp_hbm, acc_sh,
                    tbuf_v, lidx_v, crows_v, tbuf2_v, lidx2_v, crows2_v):
    c = lax.axis_index("c")
    s = lax.axis_index("s")
    nlo = c * NHN
    rlo = c * NHR

    def fill_iota(idxref, off):
        for g in range(C // 16):
            idxref[pl.ds(g * 16, 16)] = off + g * 16 + lax.iota(jnp.int32, 16)

    def init_chunk(k, carry):
        off = jnp.minimum(s * NCPT + k, CPR - 1) * C
        fill_iota(lidx_v, off)
        pltpu.sync_copy(xnp_hbm.at[pl.ds(rlo + off, C)], crows_v)
        pltpu.sync_copy(crows_v, acc_sh.at[lidx_v])
        return carry

    lax.fori_loop(0, NCPT, init_chunk, 0)
    plsc.subcore_barrier()

    ebase = s * EPT

    def localize(tbuf, lidx, count):
        for g in range(count // 16):
            v = tbuf[pl.ds(g * 16, 16)]
            loc = v - nlo
            ok = (loc >= 0) & (loc < NHN)
            lidx[pl.ds(g * 16, 16)] = jnp.where(ok, loc >> 1, TROW)

    def chunk(k, carry):
        eb = ebase + k * C
        pltpu.sync_copy(tgt_hbm.at[pl.ds(eb, C)], tbuf_v)
        pltpu.sync_copy(ct_hbm.at[pl.ds(eb, C)], crows_v)
        localize(tbuf_v, lidx_v, C)
        pltpu.sync_copy(crows_v, acc_sh.at[lidx_v], add=True)
        return carry

    lax.fori_loop(0, NCHUNK_S, chunk, 0)

    eb = ebase + NCHUNK_S * C
    pltpu.sync_copy(tgt_hbm.at[pl.ds(eb, S_TAIL)], tbuf2_v)
    pltpu.sync_copy(ct_hbm.at[pl.ds(eb, S_TAIL)], crows2_v)
    localize(tbuf2_v, lidx2_v, S_TAIL)
    pltpu.sync_copy(crows2_v, acc_sh.at[lidx2_v], add=True)

    plsc.subcore_barrier()

    def out_chunk(k, carry):
        off = jnp.minimum(s * NCPT + k, CPR - 1) * C
        fill_iota(lidx_v, off)
        pltpu.sync_copy(acc_sh.at[lidx_v], crows_v)
        pltpu.sync_copy(crows_v, outp_hbm.at[pl.ds(rlo + off, C)])
        return carry

    lax.fori_loop(0, NCPT, out_chunk, 0)


def _full(shape):
    nd = len(shape)
    return pl.BlockSpec(shape, lambda i, _nd=nd: (0,) * _nd)


def kernel(batch, X, H, edge_index, te, e3_weight, ln_gamma, ln_beta,
           Wm1, bm1, Wm2, bm2, Wx1, bx1, Wx2, bx2, Wc1, bc1, Wc2, bc2):
    batch3 = batch.astype(jnp.int32).reshape(GN, 1, NB)
    xf = X.reshape(N, XF)
    src = edge_index[0].astype(jnp.int32)
    tgt = edge_index[1].astype(jnp.int32)
    e3 = e3_weight.reshape(1, K)
    lng = ln_gamma.reshape(1, HD)
    lnb = ln_beta.reshape(1, HD)
    bm1r = bm1.reshape(1, HD)
    bm2r = bm2.reshape(1, HD)
    bx1r = bx1.reshape(1, HD)
    bx2r = bx2.reshape(1, K)
    bc1r = bc1.reshape(1, HD)
    bc2r = bc2.reshape(1, K)
    wd = Wm1[2 * HD:2 * HD + K, :]

    bspec = pl.BlockSpec((1, 1, NB), lambda i: (i, 0, 0))
    nspec = lambda w: pl.BlockSpec((NB, w), lambda i: (i, 0))

    g1 = pl.pallas_call(
        n1_body, grid=(GN,),
        in_specs=[bspec, nspec(XF)],
        out_specs=_full((B, HD)),
        out_shape=jax.ShapeDtypeStruct((B, HD), F32),
    )(batch3, xf)

    g2 = pl.pallas_call(
        n2_body, grid=(GN,),
        in_specs=[bspec, nspec(XF), _full((B, HD))],
        out_specs=_full((B, K)),
        out_shape=jax.ShapeDtypeStruct((B, K), F32),
    )(batch3, xf, g1)

    xn, s_tab, t_tab, g3 = pl.pallas_call(
        n3_body, grid=(GN,),
        in_specs=[bspec, nspec(XF), nspec(HD), _full((B, TDIM := te.shape[1])),
                  _full((B, HD)), _full((B, K)), _full((2 * HD + K + TDIM, HD)),
                  _full((1, HD)), _full((1, HD)), _full((1, HD)),
                  _full((1, K))],
        out_specs=[nspec(XF), nspec(SROW), nspec(SROW), _full((B, XF))],
        out_shape=[jax.ShapeDtypeStruct((N, XF), F32),
                   jax.ShapeDtypeStruct((N, SROW), F32),
                   jax.ShapeDtypeStruct((N, SROW), F32),
                   jax.ShapeDtypeStruct((B, XF), F32)],
    )(batch3, xf, H, te, g1, g2, Wm1, bm1r, lng, lnb, e3)

    mesh = plsc.VectorSubcoreMesh(core_axis_name="c", subcore_axis_name="s",
                                  num_cores=NC, num_subcores=NS)
    gs, gt = pl.kernel(
        gather_sc_body,
        out_type=[jax.ShapeDtypeStruct((E, SROW), F32),
                  jax.ShapeDtypeStruct((E, SROW), F32)],
        mesh=mesh,
        scratch_types=[pltpu.VMEM((C,), jnp.int32),
                       pltpu.VMEM((C, SROW), F32),
                       pltpu.SemaphoreType.DMA],
    )(s_tab, t_tab, src, tgt)

    espec = lambda w: pl.BlockSpec((EB, w), lambda i: (i, 0))
    ispec = pl.BlockSpec((1, 1, EB), lambda i: (i, 0, 0))
    src3 = src.reshape(GE, 1, EB)
    tgt3 = tgt.reshape(GE, 1, EB)
    contrib = pl.pallas_call(
        e1_body, grid=(GE,),
        in_specs=[ispec, ispec, espec(SROW), espec(SROW), _full((B, HD)),
                  _full((B, XF)), _full((K, HD)), _full((HD, HD)),
                  _full((1, HD)), _full((HD, HD)), _full((1, HD)),
                  _full((HD, K)), _full((1, K)), _full((HD, HD)),
                  _full((1, HD)), _full((HD, K)), _full((1, K))],
        out_specs=espec(SROW),
        out_shape=jax.ShapeDtypeStruct((E, SROW), F32),
    )(src3, tgt3, gs, gt, g1, g3, wd, Wm2, bm2r, Wx1, bx1r, Wx2, bx2r,
      Wc1, bc1r, Wc2, bc2r)

    xn_pad = jnp.concatenate([xn, jnp.zeros((NP - N, XF), F32)], axis=0)
    xnp = jnp.pad(xn_pad.reshape(NR, 2, XF),
                  ((0, 0), (0, 0), (0, 16))).reshape(NR, SROW)

    outp = pl.kernel(
        scatter_sc_body,
        out_type=jax.ShapeDtypeStruct((NR, SROW), F32),
        mesh=plsc.VectorSubcoreMesh(core_axis_name="c", subcore_axis_name="s",
                                    num_cores=NC, num_subcores=NS),
        scratch_types=[pltpu.VMEM_SHARED((ACC_H, SROW), F32),
                       pltpu.VMEM((C,), jnp.int32),
                       pltpu.VMEM((C,), jnp.int32),
                       pltpu.VMEM((C, SROW), F32),
                       pltpu.VMEM((S_TAIL,), jnp.int32),
                       pltpu.VMEM((S_TAIL,), jnp.int32),
                       pltpu.VMEM((S_TAIL, SROW), F32)],
    )(contrib, tgt, xnp)

    out = outp.reshape(NR, 2, 64)[:, :, :XF].reshape(NP, XF)[:N]
    return out.reshape(N, 3, K)


# speedup vs baseline: 51.9613x; 1.5099x over previous
import jax
import jax.numpy as jnp
import numpy as np
from jax import lax
from jax.experimental import pallas as pl
from jax.experimental.pallas import tpu as pltpu
from jax.experimental.pallas import tpu_sc as plsc

F32 = jnp.float32

N = 50000
E = 800000
B = 256
K = 16
HD = 64
XF = 3 * K

NB = 2000
GN = N // NB
EB = 2000
GE = E // EB

NC = 2
NS = 16
NW = NC * NS
EPW = E // NW
C = 128
NCHUNK_G = -(-EPW // C)
G_LAST = EPW - C

EPT = E // NS
NCHUNK_S = EPT // C
S_TAIL = EPT - NCHUNK_S * C

SROW = 128

_j = np.arange(XF)
_S3 = (_j[:, None] % K == np.arange(K)[None, :]).astype(np.float32)
_T3 = np.ascontiguousarray(_S3.T)
_P1 = np.zeros((XF, XF), np.float32)
_P1[((_j // K + 1) % 3) * K + _j % K, _j] = 1.0
_P2 = np.zeros((XF, XF), np.float32)
_P2[((_j // K + 2) % 3) * K + _j % K, _j] = 1.0
_SEL = np.zeros((XF, 2 * SROW), np.float32)
_SEL[_j, _j] = 1.0
_SEL[_j, SROW + 64 + _j] = 1.0

NP = 50176
NR = NP // 2
NHR = NR // NC
NHN = NHR * 2
TROW = NHR
ACC_H = NHR + 8
CPR = NHR // C
NCPT = (CPR + NS - 1) // NS


def _silu(x):
    return x / (1.0 + jnp.exp(-x))


def _onehot(b):
    return (b[:, None] == lax.broadcasted_iota(jnp.int32, (b.shape[0], B), 1)
            ).astype(F32)


def _segsum(oh, x):
    return lax.dot_general(oh, x, (((0,), (0,)), ((), ())),
                           preferred_element_type=F32)


def n1_body(b_ref, x_ref, out_ref):
    i = pl.program_id(0)
    oh = _onehot(b_ref[0, 0, :])
    sums = _segsum(oh, x_ref[...])
    cnts = _segsum(oh, jnp.ones((NB, K), F32))
    blk = jnp.concatenate([sums, cnts], axis=1)

    @pl.when(i == 0)
    def _():
        out_ref[...] = blk

    @pl.when(i > 0)
    def _():
        out_ref[...] += blk


def n2_body(b_ref, x_ref, g1_ref, out_ref):
    i = pl.program_id(0)
    oh = _onehot(b_ref[0, 0, :])
    g1 = g1_ref[...]
    cnt16 = jnp.maximum(g1[:, XF:], 1.0)
    cnt48 = jnp.concatenate([cnt16, cnt16, cnt16], axis=1)
    mean = g1[:, :XF] / cnt48
    xc = x_ref[...] - jnp.dot(oh, mean, preferred_element_type=F32)
    x0, x1, x2 = xc[:, :K], xc[:, K:2 * K], xc[:, 2 * K:]
    nrm = jnp.sqrt(x0 * x0 + x1 * x1 + x2 * x2)
    blk = _segsum(oh, nrm)

    @pl.when(i == 0)
    def _():
        out_ref[...] = blk

    @pl.when(i > 0)
    def _():
        out_ref[...] += blk


def n3_body(b_ref, x_ref, h_ref, te_ref, g1_ref, g2_ref, wm1_ref, bm1_ref,
            lng_ref, lnb_ref, e3_ref, xn_ref, s_ref, t_ref, g3_ref):
    i = pl.program_id(0)
    oh = _onehot(b_ref[0, 0, :])
    g1 = g1_ref[...]
    cnt16 = jnp.maximum(g1[:, XF:], 1.0)
    cnt48 = jnp.concatenate([cnt16, cnt16, cnt16], axis=1)
    mean = g1[:, :XF] / cnt48
    xc = x_ref[...] - jnp.dot(oh, mean, preferred_element_type=F32)
    x0, x1, x2 = xc[:, :K], xc[:, K:2 * K], xc[:, 2 * K:]
    mn = g2_ref[...] / cnt16
    denom = jnp.dot(oh, mn, preferred_element_type=F32) + 1e-5
    e3 = e3_ref[...]
    xn = jnp.concatenate(
        [e3 * x0 / denom, e3 * x1 / denom, e3 * x2 / denom], axis=1)
    xn_ref[...] = xn

    h = h_ref[...]
    mu = jnp.mean(h, axis=1, keepdims=True)
    var = jnp.mean((h - mu) ** 2, axis=1, keepdims=True)
    hn = (h - mu) / jnp.sqrt(var + 1e-5) * lng_ref[...] + lnb_ref[...]

    wm1 = wm1_ref[...]
    w_t = wm1[0:HD, :]
    w_s = wm1[HD:2 * HD, :]
    w_te = wm1[2 * HD + K:, :]
    te2 = jnp.dot(te_ref[...], w_te, preferred_element_type=F32)
    a = (jnp.dot(hn, w_s, preferred_element_type=F32)
         + jnp.dot(oh, te2, preferred_element_type=F32)
         + bm1_ref[...])
    hwt = jnp.dot(hn, w_t, preferred_element_type=F32)
    pad = jnp.zeros((NB, SROW - HD - XF), F32)
    s_ref[...] = jnp.concatenate([xn, pad, a], axis=1)
    t_ref[...] = jnp.concatenate([xn, pad, hwt], axis=1)

    blk = _segsum(oh, xn)

    @pl.when(i == 0)
    def _():
        g3_ref[...] = blk

    @pl.when(i > 0)
    def _():
        g3_ref[...] += blk


def e1_body(src_ref, tgt_ref, gs_ref, gt_ref, g1_ref, g3_ref, s3_ref, t3_ref,
            p1_ref, p2_ref, sel_ref, wd_ref, wm2_ref, bm2_ref, wx1_ref,
            bx1_ref, wx2_ref, bx2_ref, wc1_ref, bc1_ref, wc2_ref, bc2_ref,
            out_ref):
    def mm(x, w):
        return jnp.dot(x, w, preferred_element_type=F32)

    gs = gs_ref[...]
    gt = gt_ref[...]
    s3 = s3_ref[...]
    t3 = t3_ref[...]
    xs = gs[:, :XF]
    xt = gt[:, :XF]
    rall = xs - xt
    rd = mm(rall * rall, s3)

    pre1 = (gs[:, HD:] + gt[:, HD:]
            + mm(rd, wd_ref[...]))
    mij = mm(_silu(pre1), wm2_ref[...]) + bm2_ref[...]
    hx = _silu(mm(mij, wx1_ref[...]) + bx1_ref[...])
    wx = jnp.clip(mm(hx, wx2_ref[...]) + bx2_ref[...], -10.0, 10.0)
    hc = _silu(mm(mij, wc1_ref[...]) + bc1_ref[...])
    wc = jnp.clip(mm(hc, wc2_ref[...]) + bc2_ref[...], -10.0, 10.0)

    inv3 = mm(1.0 / (1.0 + jnp.sqrt(rd + 1e-8)), t3)

    g1 = g1_ref[...]
    cnt16 = jnp.maximum(g1[:, XF:], 1.0)
    cnt48 = jnp.concatenate([cnt16, cnt16, cnt16], axis=1)
    m = g3_ref[...] / cnt48
    ohs = (src_ref[0, 0, :][:, None]
           == lax.broadcasted_iota(jnp.int32, (EB, B), 1)).astype(F32)
    oht = (tgt_ref[0, 0, :][:, None]
           == lax.broadcasted_iota(jnp.int32, (EB, B), 1)).astype(F32)
    cs = xs - mm(ohs, m)
    ct = xt - mm(oht, m)
    p1 = p1_ref[...]
    p2 = p2_ref[...]
    cr = mm(cs, p1) * mm(ct, p2) - mm(cs, p2) * mm(ct, p1)
    cinv3 = mm(1.0 / (1.0 + jnp.sqrt(mm(cr * cr, s3))), t3)
    o = rall * inv3 * mm(wx, t3) + cr * cinv3 * mm(wc, t3)
    par = (tgt_ref[0, 0, :] % 2).astype(F32)[:, None]
    sel = sel_ref[...]
    out_ref[...] = (mm(o * (1.0 - par), sel[:, :SROW])
                    + mm(o * par, sel[:, SROW:]))


def gather_sc_body(s_hbm, t_hbm, src_hbm, tgt_hbm, gs_hbm, gt_hbm,
                   idx_v, rows_v, sem):
    wid = lax.axis_index("s") * NC + lax.axis_index("c")
    wbase = wid * EPW

    def chunk(j, carry):
        cb = wbase + jnp.minimum(j * C, G_LAST)
        pltpu.sync_copy(src_hbm.at[pl.ds(cb, C)], idx_v)
        pltpu.async_copy(s_hbm.at[idx_v], rows_v, sem).wait()
        pltpu.sync_copy(rows_v, gs_hbm.at[pl.ds(cb, C)])
        pltpu.sync_copy(tgt_hbm.at[pl.ds(cb, C)], idx_v)
        pltpu.async_copy(t_hbm.at[idx_v], rows_v, sem).wait()
        pltpu.sync_copy(rows_v, gt_hbm.at[pl.ds(cb, C)])
        return carry

    lax.fori_loop(0, NCHUNK_G, chunk, 0)


def scatter_sc_body(ct_hbm, tgt_hbm, xnp_hbm, outp_hbm, acc_sh,
                    tbuf_v, lidx_v, crows_v, tbuf2_v, lidx2_v, crows2_v):
    c = lax.axis_index("c")
    s = lax.axis_index("s")
    nlo = c * NHN
    rlo = c * NHR

    def fill_iota(idxref, off):
        for g in range(C // 16):
            idxref[pl.ds(g * 16, 16)] = off + g * 16 + lax.iota(jnp.int32, 16)

    def init_chunk(k, carry):
        off = jnp.minimum(s * NCPT + k, CPR - 1) * C
        fill_iota(lidx_v, off)
        pltpu.sync_copy(xnp_hbm.at[pl.ds(rlo + off, C)], crows_v)
        pltpu.sync_copy(crows_v, acc_sh.at[lidx_v])
        return carry

    lax.fori_loop(0, NCPT, init_chunk, 0)
    plsc.subcore_barrier()

    ebase = s * EPT

    def localize(tbuf, lidx, count):
        for g in range(count // 16):
            v = tbuf[pl.ds(g * 16, 16)]
            loc = v - nlo
            ok = (loc >= 0) & (loc < NHN)
            lidx[pl.ds(g * 16, 16)] = jnp.where(ok, loc >> 1, TROW)

    def chunk(k, carry):
        eb = ebase + k * C
        pltpu.sync_copy(tgt_hbm.at[pl.ds(eb, C)], tbuf_v)
        pltpu.sync_copy(ct_hbm.at[pl.ds(eb, C)], crows_v)
        localize(tbuf_v, lidx_v, C)
        pltpu.sync_copy(crows_v, acc_sh.at[lidx_v], add=True)
        return carry

    lax.fori_loop(0, NCHUNK_S, chunk, 0)

    eb = ebase + NCHUNK_S * C
    pltpu.sync_copy(tgt_hbm.at[pl.ds(eb, S_TAIL)], tbuf2_v)
    pltpu.sync_copy(ct_hbm.at[pl.ds(eb, S_TAIL)], crows2_v)
    localize(tbuf2_v, lidx2_v, S_TAIL)
    pltpu.sync_copy(crows2_v, acc_sh.at[lidx2_v], add=True)

    plsc.subcore_barrier()

    def out_chunk(k, carry):
        off = jnp.minimum(s * NCPT + k, CPR - 1) * C
        fill_iota(lidx_v, off)
        pltpu.sync_copy(acc_sh.at[lidx_v], crows_v)
        pltpu.sync_copy(crows_v, outp_hbm.at[pl.ds(rlo + off, C)])
        return carry

    lax.fori_loop(0, NCPT, out_chunk, 0)


def _full(shape):
    nd = len(shape)
    return pl.BlockSpec(shape, lambda i, _nd=nd: (0,) * _nd)


def kernel(batch, X, H, edge_index, te, e3_weight, ln_gamma, ln_beta,
           Wm1, bm1, Wm2, bm2, Wx1, bx1, Wx2, bx2, Wc1, bc1, Wc2, bc2):
    batch3 = batch.astype(jnp.int32).reshape(GN, 1, NB)
    xf = X.reshape(N, XF)
    src = edge_index[0].astype(jnp.int32)
    tgt = edge_index[1].astype(jnp.int32)
    e3 = e3_weight.reshape(1, K)
    lng = ln_gamma.reshape(1, HD)
    lnb = ln_beta.reshape(1, HD)
    bm1r = bm1.reshape(1, HD)
    bm2r = bm2.reshape(1, HD)
    bx1r = bx1.reshape(1, HD)
    bx2r = bx2.reshape(1, K)
    bc1r = bc1.reshape(1, HD)
    bc2r = bc2.reshape(1, K)
    wd = Wm1[2 * HD:2 * HD + K, :]

    bspec = pl.BlockSpec((1, 1, NB), lambda i: (i, 0, 0))
    nspec = lambda w: pl.BlockSpec((NB, w), lambda i: (i, 0))

    g1 = pl.pallas_call(
        n1_body, grid=(GN,),
        in_specs=[bspec, nspec(XF)],
        out_specs=_full((B, HD)),
        out_shape=jax.ShapeDtypeStruct((B, HD), F32),
    )(batch3, xf)

    g2 = pl.pallas_call(
        n2_body, grid=(GN,),
        in_specs=[bspec, nspec(XF), _full((B, HD))],
        out_specs=_full((B, K)),
        out_shape=jax.ShapeDtypeStruct((B, K), F32),
    )(batch3, xf, g1)

    xn, s_tab, t_tab, g3 = pl.pallas_call(
        n3_body, grid=(GN,),
        in_specs=[bspec, nspec(XF), nspec(HD), _full((B, TDIM := te.shape[1])),
                  _full((B, HD)), _full((B, K)), _full((2 * HD + K + TDIM, HD)),
                  _full((1, HD)), _full((1, HD)), _full((1, HD)),
                  _full((1, K))],
        out_specs=[nspec(XF), nspec(SROW), nspec(SROW), _full((B, XF))],
        out_shape=[jax.ShapeDtypeStruct((N, XF), F32),
                   jax.ShapeDtypeStruct((N, SROW), F32),
                   jax.ShapeDtypeStruct((N, SROW), F32),
                   jax.ShapeDtypeStruct((B, XF), F32)],
    )(batch3, xf, H, te, g1, g2, Wm1, bm1r, lng, lnb, e3)

    mesh = plsc.VectorSubcoreMesh(core_axis_name="c", subcore_axis_name="s",
                                  num_cores=NC, num_subcores=NS)
    gs, gt = pl.kernel(
        gather_sc_body,
        out_type=[jax.ShapeDtypeStruct((E, SROW), F32),
                  jax.ShapeDtypeStruct((E, SROW), F32)],
        mesh=mesh,
        scratch_types=[pltpu.VMEM((C,), jnp.int32),
                       pltpu.VMEM((C, SROW), F32),
                       pltpu.SemaphoreType.DMA],
    )(s_tab, t_tab, src, tgt)

    espec = lambda w: pl.BlockSpec((EB, w), lambda i: (i, 0))
    ispec = pl.BlockSpec((1, 1, EB), lambda i: (i, 0, 0))
    src3 = src.reshape(GE, 1, EB)
    tgt3 = tgt.reshape(GE, 1, EB)
    contrib = pl.pallas_call(
        e1_body, grid=(GE,),
        in_specs=[ispec, ispec, espec(SROW), espec(SROW), _full((B, HD)),
                  _full((B, XF)), _full((XF, K)), _full((K, XF)),
                  _full((XF, XF)), _full((XF, XF)), _full((XF, 2 * SROW)),
                  _full((K, HD)), _full((HD, HD)),
                  _full((1, HD)), _full((HD, HD)), _full((1, HD)),
                  _full((HD, K)), _full((1, K)), _full((HD, HD)),
                  _full((1, HD)), _full((HD, K)), _full((1, K))],
        out_specs=espec(SROW),
        out_shape=jax.ShapeDtypeStruct((E, SROW), F32),
    )(src3, tgt3, gs, gt, g1, g3, jnp.asarray(_S3), jnp.asarray(_T3),
      jnp.asarray(_P1), jnp.asarray(_P2), jnp.asarray(_SEL),
      wd, Wm2, bm2r, Wx1, bx1r, Wx2, bx2r, Wc1, bc1r, Wc2, bc2r)

    xn_pad = jnp.concatenate([xn, jnp.zeros((NP - N, XF), F32)], axis=0)
    xnp = jnp.pad(xn_pad.reshape(NR, 2, XF),
                  ((0, 0), (0, 0), (0, 16))).reshape(NR, SROW)

    outp = pl.kernel(
        scatter_sc_body,
        out_type=jax.ShapeDtypeStruct((NR, SROW), F32),
        mesh=plsc.VectorSubcoreMesh(core_axis_name="c", subcore_axis_name="s",
                                    num_cores=NC, num_subcores=NS),
        scratch_types=[pltpu.VMEM_SHARED((ACC_H, SROW), F32),
                       pltpu.VMEM((C,), jnp.int32),
                       pltpu.VMEM((C,), jnp.int32),
                       pltpu.VMEM((C, SROW), F32),
                       pltpu.VMEM((S_TAIL,), jnp.int32),
                       pltpu.VMEM((S_TAIL,), jnp.int32),
                       pltpu.VMEM((S_TAIL, SROW), F32)],
    )(contrib, tgt, xnp)

    out = outp.reshape(NR, 2, 64)[:, :, :XF].reshape(NP, XF)[:N]
    return out.reshape(N, 3, K)


# speedup vs baseline: 57.3490x; 1.1037x over previous
import jax
import jax.numpy as jnp
import numpy as np
from jax import lax
from jax.experimental import pallas as pl
from jax.experimental.pallas import tpu as pltpu
from jax.experimental.pallas import tpu_sc as plsc

F32 = jnp.float32

N = 50000
E = 800000
B = 256
K = 16
HD = 64
XF = 3 * K

NB = 2000
GN = N // NB
EB = 2000
GE = E // EB

NC = 2
NS = 16
NW = NC * NS
EPW = E // NW
C = 128
NCHUNK_G = -(-EPW // C)
G_LAST = EPW - C

EPT = E // NS
NCHUNK_S = EPT // C
S_TAIL = EPT - NCHUNK_S * C

SROW = 128

_j = np.arange(XF)
_S3 = (_j[:, None] % K == np.arange(K)[None, :]).astype(np.float32)
_T3 = np.ascontiguousarray(_S3.T)
_P1 = np.zeros((XF, XF), np.float32)
_P1[((_j // K + 1) % 3) * K + _j % K, _j] = 1.0
_P2 = np.zeros((XF, XF), np.float32)
_P2[((_j // K + 2) % 3) * K + _j % K, _j] = 1.0
_SEL = np.zeros((XF, 2 * SROW), np.float32)
_SEL[_j, _j] = 1.0
_SEL[_j, SROW + 64 + _j] = 1.0

NP = 50176
NR = NP // 2
NHR = NR // NC
NHN = NHR * 2
TROW = NHR
ACC_H = NHR + 8
CPR = NHR // C
NCPT = (CPR + NS - 1) // NS


def _silu(x):
    return x / (1.0 + jnp.exp(-x))


def _onehot(b):
    return (b[:, None] == lax.broadcasted_iota(jnp.int32, (b.shape[0], B), 1)
            ).astype(F32)


def _segsum(oh, x):
    return lax.dot_general(oh, x, (((0,), (0,)), ((), ())),
                           preferred_element_type=F32)


def n1_body(b_ref, x_ref, out_ref):
    i = pl.program_id(0)
    oh = _onehot(b_ref[0, 0, :])
    sums = _segsum(oh, x_ref[...])
    cnts = _segsum(oh, jnp.ones((NB, K), F32))
    blk = jnp.concatenate([sums, cnts], axis=1)

    @pl.when(i == 0)
    def _():
        out_ref[...] = blk

    @pl.when(i > 0)
    def _():
        out_ref[...] += blk


def n2_body(b_ref, x_ref, g1_ref, out_ref):
    i = pl.program_id(0)
    oh = _onehot(b_ref[0, 0, :])
    g1 = g1_ref[...]
    cnt16 = jnp.maximum(g1[:, XF:], 1.0)
    cnt48 = jnp.concatenate([cnt16, cnt16, cnt16], axis=1)
    mean = g1[:, :XF] / cnt48
    xc = x_ref[...] - jnp.dot(oh, mean, preferred_element_type=F32)
    x0, x1, x2 = xc[:, :K], xc[:, K:2 * K], xc[:, 2 * K:]
    nrm = jnp.sqrt(x0 * x0 + x1 * x1 + x2 * x2)
    blk = _segsum(oh, nrm)

    @pl.when(i == 0)
    def _():
        out_ref[...] = blk

    @pl.when(i > 0)
    def _():
        out_ref[...] += blk


def n3_body(b_ref, x_ref, h_ref, te_ref, g1_ref, g2_ref, wm1_ref, bm1_ref,
            lng_ref, lnb_ref, e3_ref, xn_ref, s_ref, t_ref, g3_ref):
    i = pl.program_id(0)
    oh = _onehot(b_ref[0, 0, :])
    g1 = g1_ref[...]
    cnt16 = jnp.maximum(g1[:, XF:], 1.0)
    cnt48 = jnp.concatenate([cnt16, cnt16, cnt16], axis=1)
    mean = g1[:, :XF] / cnt48
    xc = x_ref[...] - jnp.dot(oh, mean, preferred_element_type=F32)
    x0, x1, x2 = xc[:, :K], xc[:, K:2 * K], xc[:, 2 * K:]
    mn = g2_ref[...] / cnt16
    denom = jnp.dot(oh, mn, preferred_element_type=F32) + 1e-5
    e3 = e3_ref[...]
    xn = jnp.concatenate(
        [e3 * x0 / denom, e3 * x1 / denom, e3 * x2 / denom], axis=1)
    xn_ref[...] = xn

    h = h_ref[...]
    mu = jnp.mean(h, axis=1, keepdims=True)
    var = jnp.mean((h - mu) ** 2, axis=1, keepdims=True)
    hn = (h - mu) / jnp.sqrt(var + 1e-5) * lng_ref[...] + lnb_ref[...]

    wm1 = wm1_ref[...]
    w_t = wm1[0:HD, :]
    w_s = wm1[HD:2 * HD, :]
    w_te = wm1[2 * HD + K:, :]
    te2 = jnp.dot(te_ref[...], w_te, preferred_element_type=F32)
    a = (jnp.dot(hn, w_s, preferred_element_type=F32)
         + jnp.dot(oh, te2, preferred_element_type=F32)
         + bm1_ref[...])
    hwt = jnp.dot(hn, w_t, preferred_element_type=F32)
    pad = jnp.zeros((NB, SROW - HD - XF), F32)
    s_ref[...] = jnp.concatenate([xn, pad, a], axis=1)
    t_ref[...] = jnp.concatenate([xn, pad, hwt], axis=1)

    blk = _segsum(oh, xn)

    @pl.when(i == 0)
    def _():
        g3_ref[...] = blk

    @pl.when(i > 0)
    def _():
        g3_ref[...] += blk


def e1_body(src_ref, tgt_ref, gs_ref, gt_ref, g1_ref, g3_ref, s3_ref, t3_ref,
            p1_ref, p2_ref, sel_ref, wd_ref, wm2_ref, bm2_ref, wx1_ref,
            bx1_ref, wx2_ref, bx2_ref, wc1_ref, bc1_ref, wc2_ref, bc2_ref,
            out_ref):
    def mm(x, w):
        return jnp.dot(x, w, preferred_element_type=F32)

    gs = gs_ref[...]
    gt = gt_ref[...]
    s3 = s3_ref[...]
    t3 = t3_ref[...]
    xs = gs[:, :XF]
    xt = gt[:, :XF]
    rall = xs - xt
    rd = mm(rall * rall, s3)

    pre1 = (gs[:, HD:] + gt[:, HD:]
            + mm(rd, wd_ref[...]))
    mij = mm(_silu(pre1), wm2_ref[...]) + bm2_ref[...]
    hx = _silu(mm(mij, wx1_ref[...]) + bx1_ref[...])
    wx = jnp.clip(mm(hx, wx2_ref[...]) + bx2_ref[...], -10.0, 10.0)
    hc = _silu(mm(mij, wc1_ref[...]) + bc1_ref[...])
    wc = jnp.clip(mm(hc, wc2_ref[...]) + bc2_ref[...], -10.0, 10.0)

    inv3 = mm(1.0 / (1.0 + jnp.sqrt(rd + 1e-8)), t3)

    g1 = g1_ref[...]
    cnt16 = jnp.maximum(g1[:, XF:], 1.0)
    cnt48 = jnp.concatenate([cnt16, cnt16, cnt16], axis=1)
    m = g3_ref[...] / cnt48
    ohs = (src_ref[0, 0, :][:, None]
           == lax.broadcasted_iota(jnp.int32, (EB, B), 1)).astype(F32)
    oht = (tgt_ref[0, 0, :][:, None]
           == lax.broadcasted_iota(jnp.int32, (EB, B), 1)).astype(F32)
    cs = xs - mm(ohs, m)
    ct = xt - mm(oht, m)
    p1 = p1_ref[...]
    p2 = p2_ref[...]
    cr = mm(cs, p1) * mm(ct, p2) - mm(cs, p2) * mm(ct, p1)
    cinv3 = mm(1.0 / (1.0 + jnp.sqrt(mm(cr * cr, s3))), t3)
    o = rall * inv3 * mm(wx, t3) + cr * cinv3 * mm(wc, t3)
    par = (tgt_ref[0, 0, :] % 2).astype(F32)[:, None]
    sel = sel_ref[...]
    out_ref[...] = (mm(o * (1.0 - par), sel[:, :SROW])
                    + mm(o * par, sel[:, SROW:]))


def gather_sc_body(s_hbm, t_hbm, src_hbm, tgt_hbm, gs_hbm, gt_hbm,
                   idxs_v, idxt_v, rows_s, rows_t, sem):
    wid = lax.axis_index("s") * NC + lax.axis_index("c")
    wbase = wid * EPW

    def chunk(j, carry):
        cb = wbase + jnp.minimum(j * C, G_LAST)
        a = pltpu.async_copy(src_hbm.at[pl.ds(cb, C)], idxs_v, sem)
        b = pltpu.async_copy(tgt_hbm.at[pl.ds(cb, C)], idxt_v, sem)
        a.wait()
        b.wait()
        a = pltpu.async_copy(s_hbm.at[idxs_v], rows_s, sem)
        b = pltpu.async_copy(t_hbm.at[idxt_v], rows_t, sem)
        a.wait()
        b.wait()
        a = pltpu.async_copy(rows_s, gs_hbm.at[pl.ds(cb, C)], sem)
        b = pltpu.async_copy(rows_t, gt_hbm.at[pl.ds(cb, C)], sem)
        a.wait()
        b.wait()
        return carry

    lax.fori_loop(0, NCHUNK_G, chunk, 0)


def scatter_sc_body(ct_hbm, tgt_hbm, xnp_hbm, outp_hbm, acc_sh,
                    tbuf_v, lidx_v, crows_v, tbuf2_v, lidx2_v, crows2_v,
                    sem):
    c = lax.axis_index("c")
    s = lax.axis_index("s")
    nlo = c * NHN
    rlo = c * NHR

    def fill_iota(idxref, off):
        for g in range(C // 16):
            idxref[pl.ds(g * 16, 16)] = off + g * 16 + lax.iota(jnp.int32, 16)

    def init_chunk(k, carry):
        off = jnp.minimum(s * NCPT + k, CPR - 1) * C
        fill_iota(lidx_v, off)
        pltpu.sync_copy(xnp_hbm.at[pl.ds(rlo + off, C)], crows_v)
        pltpu.sync_copy(crows_v, acc_sh.at[lidx_v])
        return carry

    lax.fori_loop(0, NCPT, init_chunk, 0)
    plsc.subcore_barrier()

    ebase = s * EPT

    def localize(tbuf, lidx, count):
        for g in range(count // 16):
            v = tbuf[pl.ds(g * 16, 16)]
            loc = v - nlo
            ok = (loc >= 0) & (loc < NHN)
            lidx[pl.ds(g * 16, 16)] = jnp.where(ok, loc >> 1, TROW)

    def chunk(k, carry):
        eb = ebase + k * C
        a = pltpu.async_copy(tgt_hbm.at[pl.ds(eb, C)], tbuf_v, sem)
        b = pltpu.async_copy(ct_hbm.at[pl.ds(eb, C)], crows_v, sem)
        a.wait()
        b.wait()
        localize(tbuf_v, lidx_v, C)
        pltpu.sync_copy(crows_v, acc_sh.at[lidx_v], add=True)
        return carry

    lax.fori_loop(0, NCHUNK_S, chunk, 0)

    eb = ebase + NCHUNK_S * C
    pltpu.sync_copy(tgt_hbm.at[pl.ds(eb, S_TAIL)], tbuf2_v)
    pltpu.sync_copy(ct_hbm.at[pl.ds(eb, S_TAIL)], crows2_v)
    localize(tbuf2_v, lidx2_v, S_TAIL)
    pltpu.sync_copy(crows2_v, acc_sh.at[lidx2_v], add=True)

    plsc.subcore_barrier()

    def out_chunk(k, carry):
        off = jnp.minimum(s * NCPT + k, CPR - 1) * C
        fill_iota(lidx_v, off)
        pltpu.sync_copy(acc_sh.at[lidx_v], crows_v)
        pltpu.sync_copy(crows_v, outp_hbm.at[pl.ds(rlo + off, C)])
        return carry

    lax.fori_loop(0, NCPT, out_chunk, 0)


def _full(shape):
    nd = len(shape)
    return pl.BlockSpec(shape, lambda i, _nd=nd: (0,) * _nd)


def kernel(batch, X, H, edge_index, te, e3_weight, ln_gamma, ln_beta,
           Wm1, bm1, Wm2, bm2, Wx1, bx1, Wx2, bx2, Wc1, bc1, Wc2, bc2):
    batch3 = batch.astype(jnp.int32).reshape(GN, 1, NB)
    xf = X.reshape(N, XF)
    src = edge_index[0].astype(jnp.int32)
    tgt = edge_index[1].astype(jnp.int32)
    e3 = e3_weight.reshape(1, K)
    lng = ln_gamma.reshape(1, HD)
    lnb = ln_beta.reshape(1, HD)
    bm1r = bm1.reshape(1, HD)
    bm2r = bm2.reshape(1, HD)
    bx1r = bx1.reshape(1, HD)
    bx2r = bx2.reshape(1, K)
    bc1r = bc1.reshape(1, HD)
    bc2r = bc2.reshape(1, K)
    wd = Wm1[2 * HD:2 * HD + K, :]

    bspec = pl.BlockSpec((1, 1, NB), lambda i: (i, 0, 0))
    nspec = lambda w: pl.BlockSpec((NB, w), lambda i: (i, 0))

    g1 = pl.pallas_call(
        n1_body, grid=(GN,),
        in_specs=[bspec, nspec(XF)],
        out_specs=_full((B, HD)),
        out_shape=jax.ShapeDtypeStruct((B, HD), F32),
    )(batch3, xf)

    g2 = pl.pallas_call(
        n2_body, grid=(GN,),
        in_specs=[bspec, nspec(XF), _full((B, HD))],
        out_specs=_full((B, K)),
        out_shape=jax.ShapeDtypeStruct((B, K), F32),
    )(batch3, xf, g1)

    xn, s_tab, t_tab, g3 = pl.pallas_call(
        n3_body, grid=(GN,),
        in_specs=[bspec, nspec(XF), nspec(HD), _full((B, TDIM := te.shape[1])),
                  _full((B, HD)), _full((B, K)), _full((2 * HD + K + TDIM, HD)),
                  _full((1, HD)), _full((1, HD)), _full((1, HD)),
                  _full((1, K))],
        out_specs=[nspec(XF), nspec(SROW), nspec(SROW), _full((B, XF))],
        out_shape=[jax.ShapeDtypeStruct((N, XF), F32),
                   jax.ShapeDtypeStruct((N, SROW), F32),
                   jax.ShapeDtypeStruct((N, SROW), F32),
                   jax.ShapeDtypeStruct((B, XF), F32)],
    )(batch3, xf, H, te, g1, g2, Wm1, bm1r, lng, lnb, e3)

    mesh = plsc.VectorSubcoreMesh(core_axis_name="c", subcore_axis_name="s",
                                  num_cores=NC, num_subcores=NS)
    gs, gt = pl.kernel(
        gather_sc_body,
        out_type=[jax.ShapeDtypeStruct((E, SROW), F32),
                  jax.ShapeDtypeStruct((E, SROW), F32)],
        mesh=mesh,
        scratch_types=[pltpu.VMEM((C,), jnp.int32),
                       pltpu.VMEM((C,), jnp.int32),
                       pltpu.VMEM((C, SROW), F32),
                       pltpu.VMEM((C, SROW), F32),
                       pltpu.SemaphoreType.DMA],
    )(s_tab, t_tab, src, tgt)

    espec = lambda w: pl.BlockSpec((EB, w), lambda i: (i, 0))
    ispec = pl.BlockSpec((1, 1, EB), lambda i: (i, 0, 0))
    src3 = src.reshape(GE, 1, EB)
    tgt3 = tgt.reshape(GE, 1, EB)
    contrib = pl.pallas_call(
        e1_body, grid=(GE,),
        in_specs=[ispec, ispec, espec(SROW), espec(SROW), _full((B, HD)),
                  _full((B, XF)), _full((XF, K)), _full((K, XF)),
                  _full((XF, XF)), _full((XF, XF)), _full((XF, 2 * SROW)),
                  _full((K, HD)), _full((HD, HD)),
                  _full((1, HD)), _full((HD, HD)), _full((1, HD)),
                  _full((HD, K)), _full((1, K)), _full((HD, HD)),
                  _full((1, HD)), _full((HD, K)), _full((1, K))],
        out_specs=espec(SROW),
        out_shape=jax.ShapeDtypeStruct((E, SROW), F32),
    )(src3, tgt3, gs, gt, g1, g3, jnp.asarray(_S3), jnp.asarray(_T3),
      jnp.asarray(_P1), jnp.asarray(_P2), jnp.asarray(_SEL),
      wd, Wm2, bm2r, Wx1, bx1r, Wx2, bx2r, Wc1, bc1r, Wc2, bc2r)

    xn_pad = jnp.concatenate([xn, jnp.zeros((NP - N, XF), F32)], axis=0)
    xnp = jnp.pad(xn_pad.reshape(NR, 2, XF),
                  ((0, 0), (0, 0), (0, 16))).reshape(NR, SROW)

    outp = pl.kernel(
        scatter_sc_body,
        out_type=jax.ShapeDtypeStruct((NR, SROW), F32),
        mesh=plsc.VectorSubcoreMesh(core_axis_name="c", subcore_axis_name="s",
                                    num_cores=NC, num_subcores=NS),
        scratch_types=[pltpu.VMEM_SHARED((ACC_H, SROW), F32),
                       pltpu.VMEM((C,), jnp.int32),
                       pltpu.VMEM((C,), jnp.int32),
                       pltpu.VMEM((C, SROW), F32),
                       pltpu.VMEM((S_TAIL,), jnp.int32),
                       pltpu.VMEM((S_TAIL,), jnp.int32),
                       pltpu.VMEM((S_TAIL, SROW), F32),
                       pltpu.SemaphoreType.DMA],
    )(contrib, tgt, xnp)

    out = outp.reshape(NR, 2, 64)[:, :, :XF].reshape(NP, XF)[:N]
    return out.reshape(N, 3, K)


# speedup vs baseline: 59.3133x; 1.0343x over previous
import jax
import jax.numpy as jnp
import numpy as np
from jax import lax
from jax.experimental import pallas as pl
from jax.experimental.pallas import tpu as pltpu
from jax.experimental.pallas import tpu_sc as plsc

F32 = jnp.float32

N = 50000
E = 800000
B = 256
K = 16
HD = 64
XF = 3 * K

NB = 2000
GN = N // NB
EB = 4000
GE = E // EB

NC = 2
NS = 16
NW = NC * NS
EPW = E // NW
C = 128
NCHUNK_G = -(-EPW // C)
G_LAST = EPW - C

EPT = E // NS
NCHUNK_S = EPT // C
S_TAIL = EPT - NCHUNK_S * C

SROW = 128

_j = np.arange(XF)
_S3 = (_j[:, None] % K == np.arange(K)[None, :]).astype(np.float32)
_T3 = np.ascontiguousarray(_S3.T)
_P1 = np.zeros((XF, XF), np.float32)
_P1[((_j // K + 1) % 3) * K + _j % K, _j] = 1.0
_P2 = np.zeros((XF, XF), np.float32)
_P2[((_j // K + 2) % 3) * K + _j % K, _j] = 1.0
_SEL = np.zeros((XF, 2 * SROW), np.float32)
_SEL[_j, _j] = 1.0
_SEL[_j, SROW + 64 + _j] = 1.0

NP = 50176
NR = NP // 2
NHR = NR // NC
NHN = NHR * 2
TROW = NHR
ACC_H = NHR + 8
CPR = NHR // C
NCPT = (CPR + NS - 1) // NS


def _silu(x):
    return x / (1.0 + jnp.exp(-x))


def _onehot(b):
    return (b[:, None] == lax.broadcasted_iota(jnp.int32, (b.shape[0], B), 1)
            ).astype(F32)


def _segsum(oh, x):
    return lax.dot_general(oh, x, (((0,), (0,)), ((), ())),
                           preferred_element_type=F32)


def n1_body(b_ref, x_ref, out_ref):
    i = pl.program_id(0)
    oh = _onehot(b_ref[0, 0, :])
    sums = _segsum(oh, x_ref[...])
    cnts = _segsum(oh, jnp.ones((NB, K), F32))
    blk = jnp.concatenate([sums, cnts], axis=1)

    @pl.when(i == 0)
    def _():
        out_ref[...] = blk

    @pl.when(i > 0)
    def _():
        out_ref[...] += blk


def n2_body(b_ref, x_ref, g1_ref, out_ref):
    i = pl.program_id(0)
    oh = _onehot(b_ref[0, 0, :])
    g1 = g1_ref[...]
    cnt16 = jnp.maximum(g1[:, XF:], 1.0)
    cnt48 = jnp.concatenate([cnt16, cnt16, cnt16], axis=1)
    mean = g1[:, :XF] / cnt48
    xc = x_ref[...] - jnp.dot(oh, mean, preferred_element_type=F32)
    x0, x1, x2 = xc[:, :K], xc[:, K:2 * K], xc[:, 2 * K:]
    nrm = jnp.sqrt(x0 * x0 + x1 * x1 + x2 * x2)
    blk = _segsum(oh, nrm)

    @pl.when(i == 0)
    def _():
        out_ref[...] = blk

    @pl.when(i > 0)
    def _():
        out_ref[...] += blk


def n3_body(b_ref, x_ref, h_ref, te_ref, g1_ref, g2_ref, wm1_ref, bm1_ref,
            lng_ref, lnb_ref, e3_ref, xn_ref, s_ref, t_ref, g3_ref):
    i = pl.program_id(0)
    oh = _onehot(b_ref[0, 0, :])
    g1 = g1_ref[...]
    cnt16 = jnp.maximum(g1[:, XF:], 1.0)
    cnt48 = jnp.concatenate([cnt16, cnt16, cnt16], axis=1)
    mean = g1[:, :XF] / cnt48
    xc = x_ref[...] - jnp.dot(oh, mean, preferred_element_type=F32)
    x0, x1, x2 = xc[:, :K], xc[:, K:2 * K], xc[:, 2 * K:]
    mn = g2_ref[...] / cnt16
    denom = jnp.dot(oh, mn, preferred_element_type=F32) + 1e-5
    e3 = e3_ref[...]
    xn = jnp.concatenate(
        [e3 * x0 / denom, e3 * x1 / denom, e3 * x2 / denom], axis=1)
    xn_ref[...] = xn

    h = h_ref[...]
    mu = jnp.mean(h, axis=1, keepdims=True)
    var = jnp.mean((h - mu) ** 2, axis=1, keepdims=True)
    hn = (h - mu) / jnp.sqrt(var + 1e-5) * lng_ref[...] + lnb_ref[...]

    wm1 = wm1_ref[...]
    w_t = wm1[0:HD, :]
    w_s = wm1[HD:2 * HD, :]
    w_te = wm1[2 * HD + K:, :]
    te2 = jnp.dot(te_ref[...], w_te, preferred_element_type=F32)
    a = (jnp.dot(hn, w_s, preferred_element_type=F32)
         + jnp.dot(oh, te2, preferred_element_type=F32)
         + bm1_ref[...])
    hwt = jnp.dot(hn, w_t, preferred_element_type=F32)
    pad = jnp.zeros((NB, SROW - HD - XF), F32)
    s_ref[...] = jnp.concatenate([xn, pad, a], axis=1)
    t_ref[...] = jnp.concatenate([xn, pad, hwt], axis=1)

    blk = _segsum(oh, xn)

    @pl.when(i == 0)
    def _():
        g3_ref[...] = blk

    @pl.when(i > 0)
    def _():
        g3_ref[...] += blk


def e1_body(src_ref, tgt_ref, gs_ref, gt_ref, g1_ref, g3_ref, s3_ref, t3_ref,
            p1_ref, p2_ref, sel_ref, wd_ref, wm2_ref, bm2_ref, wx1_ref,
            bx1_ref, wx2_ref, bx2_ref, wc1_ref, bc1_ref, wc2_ref, bc2_ref,
            out_ref):
    def mm(x, w):
        return jnp.dot(x, w, preferred_element_type=F32)

    gs = gs_ref[...]
    gt = gt_ref[...]
    s3 = s3_ref[...]
    t3 = t3_ref[...]
    xs = gs[:, :XF]
    xt = gt[:, :XF]
    rall = xs - xt
    rd = mm(rall * rall, s3)

    pre1 = (gs[:, HD:] + gt[:, HD:]
            + mm(rd, wd_ref[...]))
    mij = mm(_silu(pre1), wm2_ref[...]) + bm2_ref[...]
    hx = _silu(mm(mij, wx1_ref[...]) + bx1_ref[...])
    wx = jnp.clip(mm(hx, wx2_ref[...]) + bx2_ref[...], -10.0, 10.0)
    hc = _silu(mm(mij, wc1_ref[...]) + bc1_ref[...])
    wc = jnp.clip(mm(hc, wc2_ref[...]) + bc2_ref[...], -10.0, 10.0)

    inv3 = mm(1.0 / (1.0 + jnp.sqrt(rd + 1e-8)), t3)

    g1 = g1_ref[...]
    cnt16 = jnp.maximum(g1[:, XF:], 1.0)
    cnt48 = jnp.concatenate([cnt16, cnt16, cnt16], axis=1)
    m = g3_ref[...] / cnt48
    ohs = (src_ref[0, 0, :][:, None]
           == lax.broadcasted_iota(jnp.int32, (EB, B), 1)).astype(F32)
    oht = (tgt_ref[0, 0, :][:, None]
           == lax.broadcasted_iota(jnp.int32, (EB, B), 1)).astype(F32)
    cs = xs - mm(ohs, m)
    ct = xt - mm(oht, m)
    p1 = p1_ref[...]
    p2 = p2_ref[...]
    cr = mm(cs, p1) * mm(ct, p2) - mm(cs, p2) * mm(ct, p1)
    cinv3 = mm(1.0 / (1.0 + jnp.sqrt(mm(cr * cr, s3))), t3)
    o = rall * inv3 * mm(wx, t3) + cr * cinv3 * mm(wc, t3)
    par = (tgt_ref[0, 0, :] % 2).astype(F32)[:, None]
    sel = sel_ref[...]
    out_ref[...] = (mm(o * (1.0 - par), sel[:, :SROW])
                    + mm(o * par, sel[:, SROW:]))


def gather_sc_body(s_hbm, t_hbm, src_hbm, tgt_hbm, gs_hbm, gt_hbm,
                   idxs_v, idxt_v, rows_s, rows_t, sem):
    wid = lax.axis_index("s") * NC + lax.axis_index("c")
    wbase = wid * EPW

    def chunk(j, carry):
        cb = wbase + jnp.minimum(j * C, G_LAST)
        a = pltpu.async_copy(src_hbm.at[pl.ds(cb, C)], idxs_v, sem)
        b = pltpu.async_copy(tgt_hbm.at[pl.ds(cb, C)], idxt_v, sem)
        a.wait()
        b.wait()
        a = pltpu.async_copy(s_hbm.at[idxs_v], rows_s, sem)
        b = pltpu.async_copy(t_hbm.at[idxt_v], rows_t, sem)
        a.wait()
        b.wait()
        a = pltpu.async_copy(rows_s, gs_hbm.at[pl.ds(cb, C)], sem)
        b = pltpu.async_copy(rows_t, gt_hbm.at[pl.ds(cb, C)], sem)
        a.wait()
        b.wait()
        return carry

    lax.fori_loop(0, NCHUNK_G, chunk, 0)


def scatter_sc_body(ct_hbm, tgt_hbm, xnp_hbm, outp_hbm, acc_sh,
                    tbuf_v, lidx_v, crows_v, tbuf2_v, lidx2_v, crows2_v,
                    sem):
    c = lax.axis_index("c")
    s = lax.axis_index("s")
    nlo = c * NHN
    rlo = c * NHR

    def fill_iota(idxref, off):
        for g in range(C // 16):
            idxref[pl.ds(g * 16, 16)] = off + g * 16 + lax.iota(jnp.int32, 16)

    def init_chunk(k, carry):
        off = jnp.minimum(s * NCPT + k, CPR - 1) * C
        fill_iota(lidx_v, off)
        pltpu.sync_copy(xnp_hbm.at[pl.ds(rlo + off, C)], crows_v)
        pltpu.sync_copy(crows_v, acc_sh.at[lidx_v])
        return carry

    lax.fori_loop(0, NCPT, init_chunk, 0)
    plsc.subcore_barrier()

    ebase = s * EPT

    def localize(tbuf, lidx, count):
        for g in range(count // 16):
            v = tbuf[pl.ds(g * 16, 16)]
            loc = v - nlo
            ok = (loc >= 0) & (loc < NHN)
            lidx[pl.ds(g * 16, 16)] = jnp.where(ok, loc >> 1, TROW)

    def chunk(k, carry):
        eb = ebase + k * C
        a = pltpu.async_copy(tgt_hbm.at[pl.ds(eb, C)], tbuf_v, sem)
        b = pltpu.async_copy(ct_hbm.at[pl.ds(eb, C)], crows_v, sem)
        a.wait()
        b.wait()
        localize(tbuf_v, lidx_v, C)
        pltpu.sync_copy(crows_v, acc_sh.at[lidx_v], add=True)
        return carry

    lax.fori_loop(0, NCHUNK_S, chunk, 0)

    eb = ebase + NCHUNK_S * C
    pltpu.sync_copy(tgt_hbm.at[pl.ds(eb, S_TAIL)], tbuf2_v)
    pltpu.sync_copy(ct_hbm.at[pl.ds(eb, S_TAIL)], crows2_v)
    localize(tbuf2_v, lidx2_v, S_TAIL)
    pltpu.sync_copy(crows2_v, acc_sh.at[lidx2_v], add=True)

    plsc.subcore_barrier()

    def out_chunk(k, carry):
        off = jnp.minimum(s * NCPT + k, CPR - 1) * C
        fill_iota(lidx_v, off)
        pltpu.sync_copy(acc_sh.at[lidx_v], crows_v)
        pltpu.sync_copy(crows_v, outp_hbm.at[pl.ds(rlo + off, C)])
        return carry

    lax.fori_loop(0, NCPT, out_chunk, 0)


def _full(shape):
    nd = len(shape)
    return pl.BlockSpec(shape, lambda i, _nd=nd: (0,) * _nd)


def kernel(batch, X, H, edge_index, te, e3_weight, ln_gamma, ln_beta,
           Wm1, bm1, Wm2, bm2, Wx1, bx1, Wx2, bx2, Wc1, bc1, Wc2, bc2):
    batch3 = batch.astype(jnp.int32).reshape(GN, 1, NB)
    xf = X.reshape(N, XF)
    src = edge_index[0].astype(jnp.int32)
    tgt = edge_index[1].astype(jnp.int32)
    e3 = e3_weight.reshape(1, K)
    lng = ln_gamma.reshape(1, HD)
    lnb = ln_beta.reshape(1, HD)
    bm1r = bm1.reshape(1, HD)
    bm2r = bm2.reshape(1, HD)
    bx1r = bx1.reshape(1, HD)
    bx2r = bx2.reshape(1, K)
    bc1r = bc1.reshape(1, HD)
    bc2r = bc2.reshape(1, K)
    wd = Wm1[2 * HD:2 * HD + K, :]

    bspec = pl.BlockSpec((1, 1, NB), lambda i: (i, 0, 0))
    nspec = lambda w: pl.BlockSpec((NB, w), lambda i: (i, 0))

    g1 = pl.pallas_call(
        n1_body, grid=(GN,),
        in_specs=[bspec, nspec(XF)],
        out_specs=_full((B, HD)),
        out_shape=jax.ShapeDtypeStruct((B, HD), F32),
    )(batch3, xf)

    g2 = pl.pallas_call(
        n2_body, grid=(GN,),
        in_specs=[bspec, nspec(XF), _full((B, HD))],
        out_specs=_full((B, K)),
        out_shape=jax.ShapeDtypeStruct((B, K), F32),
    )(batch3, xf, g1)

    xn, s_tab, t_tab, g3 = pl.pallas_call(
        n3_body, grid=(GN,),
        in_specs=[bspec, nspec(XF), nspec(HD), _full((B, TDIM := te.shape[1])),
                  _full((B, HD)), _full((B, K)), _full((2 * HD + K + TDIM, HD)),
                  _full((1, HD)), _full((1, HD)), _full((1, HD)),
                  _full((1, K))],
        out_specs=[nspec(XF), nspec(SROW), nspec(SROW), _full((B, XF))],
        out_shape=[jax.ShapeDtypeStruct((N, XF), F32),
                   jax.ShapeDtypeStruct((N, SROW), F32),
                   jax.ShapeDtypeStruct((N, SROW), F32),
                   jax.ShapeDtypeStruct((B, XF), F32)],
    )(batch3, xf, H, te, g1, g2, Wm1, bm1r, lng, lnb, e3)

    mesh = plsc.VectorSubcoreMesh(core_axis_name="c", subcore_axis_name="s",
                                  num_cores=NC, num_subcores=NS)
    gs, gt = pl.kernel(
        gather_sc_body,
        out_type=[jax.ShapeDtypeStruct((E, SROW), F32),
                  jax.ShapeDtypeStruct((E, SROW), F32)],
        mesh=mesh,
        scratch_types=[pltpu.VMEM((C,), jnp.int32),
                       pltpu.VMEM((C,), jnp.int32),
                       pltpu.VMEM((C, SROW), F32),
                       pltpu.VMEM((C, SROW), F32),
                       pltpu.SemaphoreType.DMA],
    )(s_tab, t_tab, src, tgt)

    espec = lambda w: pl.BlockSpec((EB, w), lambda i: (i, 0))
    ispec = pl.BlockSpec((1, 1, EB), lambda i: (i, 0, 0))
    src3 = src.reshape(GE, 1, EB)
    tgt3 = tgt.reshape(GE, 1, EB)
    contrib = pl.pallas_call(
        e1_body, grid=(GE,),
        in_specs=[ispec, ispec, espec(SROW), espec(SROW), _full((B, HD)),
                  _full((B, XF)), _full((XF, K)), _full((K, XF)),
                  _full((XF, XF)), _full((XF, XF)), _full((XF, 2 * SROW)),
                  _full((K, HD)), _full((HD, HD)),
                  _full((1, HD)), _full((HD, HD)), _full((1, HD)),
                  _full((HD, K)), _full((1, K)), _full((HD, HD)),
                  _full((1, HD)), _full((HD, K)), _full((1, K))],
        out_specs=espec(SROW),
        out_shape=jax.ShapeDtypeStruct((E, SROW), F32),
    )(src3, tgt3, gs, gt, g1, g3, jnp.asarray(_S3), jnp.asarray(_T3),
      jnp.asarray(_P1), jnp.asarray(_P2), jnp.asarray(_SEL),
      wd, Wm2, bm2r, Wx1, bx1r, Wx2, bx2r, Wc1, bc1r, Wc2, bc2r)

    xn_pad = jnp.concatenate([xn, jnp.zeros((NP - N, XF), F32)], axis=0)
    xnp = jnp.pad(xn_pad.reshape(NR, 2, XF),
                  ((0, 0), (0, 0), (0, 16))).reshape(NR, SROW)

    outp = pl.kernel(
        scatter_sc_body,
        out_type=jax.ShapeDtypeStruct((NR, SROW), F32),
        mesh=plsc.VectorSubcoreMesh(core_axis_name="c", subcore_axis_name="s",
                                    num_cores=NC, num_subcores=NS),
        scratch_types=[pltpu.VMEM_SHARED((ACC_H, SROW), F32),
                       pltpu.VMEM((C,), jnp.int32),
                       pltpu.VMEM((C,), jnp.int32),
                       pltpu.VMEM((C, SROW), F32),
                       pltpu.VMEM((S_TAIL,), jnp.int32),
                       pltpu.VMEM((S_TAIL,), jnp.int32),
                       pltpu.VMEM((S_TAIL, SROW), F32),
                       pltpu.SemaphoreType.DMA],
    )(contrib, tgt, xnp)

    out = outp.reshape(NR, 2, 64)[:, :, :XF].reshape(NP, XF)[:N]
    return out.reshape(N, 3, K)
